# Initial kernel scaffold; baseline (speedup 1.0000x reference)
import functools

import jax
import jax.numpy as jnp
from jax import lax
from jax.experimental import pallas as pl
from jax.experimental.pallas import tpu as pltpu
from jax.experimental.pallas import tpu_sc as plsc

_ALPHA = 0.25
_K = 100000

_B, _C, _H, _W = 8, 4, 512, 512
_N = _B * _C * _H * _W
_NW = 32
_PER_W = _N // _NW
_CHUNK = 16384
_NCHUNK = _PER_W // _CHUNK
_NB = 4096


def _loss_body(pred_ref, tgt_ref, mask_ref, key_ref):
    x = pred_ref[0, 0]
    t = tgt_ref[0, 0].astype(jnp.float32)
    e = jnp.exp(-jnp.abs(x))
    p = jnp.where(x >= 0.0, 1.0 / (1.0 + e), e / (1.0 + e))
    ce = jnp.maximum(x, 0.0) - x * t + jnp.log1p(e)
    p_t = p * t + (1.0 - p) * (1.0 - t)
    one_m = 1.0 - p_t
    loss = ce * (one_m * one_m)
    alpha_t = _ALPHA * t + (1.0 - _ALPHA) * (1.0 - t)
    loss = alpha_t * loss + 0.0
    key = lax.bitcast_convert_type(loss, jnp.int32)
    key_ref[0, 0] = jnp.where(mask_ref[...] == 0, key, -1)


def _loss_keys(predictions, targets, mask_plane, interpret=False):
    return pl.pallas_call(
        _loss_body,
        grid=(_B, _C),
        in_specs=[
            pl.BlockSpec((1, 1, _H, _W), lambda b, c: (b, c, 0, 0)),
            pl.BlockSpec((1, 1, _H, _W), lambda b, c: (b, c + 1, 0, 0)),
            pl.BlockSpec((_H, _W), lambda b, c: (0, 0)),
        ],
        out_specs=pl.BlockSpec((1, 1, _H, _W), lambda b, c: (b, c, 0, 0)),
        out_shape=jax.ShapeDtypeStruct((_B, _C, _H, _W), jnp.int32),
        interpret=interpret,
    )(predictions, targets, mask_plane)


def _make_hist_pass(stage, interpret=False):
    mesh = plsc.VectorSubcoreMesh(core_axis_name="c", subcore_axis_name="s")

    @functools.partial(
        pl.kernel,
        out_type=(
            jax.ShapeDtypeStruct((_NW, _NB), jnp.int32),
            jax.ShapeDtypeStruct((_NW, _NB), jnp.float32),
        ),
        mesh=mesh,
        scratch_types=[
            pltpu.VMEM((_CHUNK,), jnp.int32),
            pltpu.VMEM((_CHUNK,), jnp.int32),
            pltpu.VMEM((_NB,), jnp.int32),
            pltpu.VMEM((_NB,), jnp.float32),
            pltpu.VMEM((16,), jnp.int32),
            pltpu.SemaphoreType.DMA,
            pltpu.SemaphoreType.DMA,
        ],
        interpret=interpret,
    )
    def hist(keys_hbm, pfx_hbm, cnt_hbm, sum_hbm,
             buf0, buf1, cnt, sm, pfxv, sem0, sem1):
        wid = lax.axis_index("s") * 2 + lax.axis_index("c")
        base = wid * _PER_W

        zero16i = jnp.zeros((16,), jnp.int32)
        zero16f = jnp.zeros((16,), jnp.float32)

        def zbody(i, carry):
            cnt[pl.ds(i * 16, 16)] = zero16i
            sm[pl.ds(i * 16, 16)] = zero16f
            return carry

        lax.fori_loop(0, _NB // 16, zbody, 0)

        pltpu.sync_copy(pfx_hbm, pfxv)
        pfx = pfxv[...]
        ones = jnp.ones((16,), jnp.int32)

        bufs = (buf0, buf1)
        sems = (sem0, sem1)
        copies = [None, None]
        copies[0] = pltpu.async_copy(
            keys_hbm.at[pl.ds(base, _CHUNK)], buf0, sem0)
        for c in range(_NCHUNK):
            if c + 1 < _NCHUNK:
                nxt = (c + 1) % 2
                copies[nxt] = pltpu.async_copy(
                    keys_hbm.at[pl.ds(base + (c + 1) * _CHUNK, _CHUNK)],
                    bufs[nxt], sems[nxt])
            copies[c % 2].wait()
            buf = bufs[c % 2]

            def body(i, carry):
                key = buf[pl.ds(i * 16, 16)]
                if stage == 0:
                    valid = key >= 0
                    bucket = lax.shift_right_logical(key, 19)
                else:
                    valid = jnp.logical_and(
                        key >= 0, lax.shift_right_logical(key, 19) == pfx)
                    bucket = jnp.bitwise_and(
                        lax.shift_right_logical(key, 7), _NB - 1)
                bucket = jnp.where(valid, bucket, 0)
                plsc.addupdate_scatter(cnt, [bucket], ones, mask=valid)
                val = jnp.where(valid, plsc.bitcast(key, jnp.float32), 0.0)
                plsc.addupdate_scatter(sm, [bucket], val, mask=valid)
                return carry

            lax.fori_loop(0, _CHUNK // 16, body, 0)

        pltpu.sync_copy(cnt, cnt_hbm.at[wid])
        pltpu.sync_copy(sm, sum_hbm.at[wid])

    return hist


_hist_pass0 = _make_hist_pass(0)
_hist_pass1 = _make_hist_pass(1)


def _select(cnt, sm, need):
    rc = jnp.cumsum(cnt[::-1])[::-1]
    ca = rc - cnt
    cross = jnp.logical_and(ca < need, rc >= need)
    b = jnp.argmax(cross)
    rs = jnp.cumsum(sm[::-1])[::-1]
    sa = rs - sm
    return b, need - ca[b], sa[b]


def kernel(predictions, targets, batch_idx):
    mask_plane = lax.dynamic_index_in_dim(
        targets, batch_idx, axis=0, keepdims=False)[0]
    keys = _loss_keys(predictions, targets, mask_plane).reshape(_N)

    pfx0 = jnp.zeros((16,), jnp.int32)
    cnt_t, sum_t = _hist_pass0(keys, pfx0)
    cnt1 = cnt_t.sum(0)
    sum1 = sum_t.sum(0)
    b1, k1, s1 = _select(cnt1, sum1, _K)

    pfx1 = jnp.full((16,), b1, jnp.int32)
    cnt_t2, sum_t2 = _hist_pass1(keys, pfx1)
    cnt2 = cnt_t2.sum(0)
    sum2 = sum_t2.sum(0)
    b2, k2, s2 = _select(cnt2, sum2, k1)

    avg = sum2[b2] / cnt2[b2].astype(jnp.float32)
    res = (s1 + s2 + k2.astype(jnp.float32) * avg) / jnp.float32(_K)
    total = cnt1.sum()
    return jnp.where(total >= _K, res, -jnp.inf).astype(jnp.float32)


# speedup vs baseline: 25.3443x; 25.3443x over previous
import functools

import jax
import jax.numpy as jnp
from jax import lax
from jax.experimental import pallas as pl
from jax.experimental.pallas import tpu as pltpu
from jax.experimental.pallas import tpu_sc as plsc

_ALPHA = 0.25
_K = 100000

_B, _C, _H, _W = 8, 4, 512, 512
_N = _B * _C * _H * _W
_NW = 32
_PER_W = _N // _NW
_CHUNK = 16384
_NCHUNK = _PER_W // _CHUNK
_NB = 4096


def _loss_body(pred_ref, tgt_ref, mask_ref, key_ref):
    x = pred_ref[0, 0]
    t = tgt_ref[0, 0].astype(jnp.float32)
    e = jnp.exp(-jnp.abs(x))
    p = jnp.where(x >= 0.0, 1.0 / (1.0 + e), e / (1.0 + e))
    ce = jnp.maximum(x, 0.0) - x * t + jnp.log1p(e)
    p_t = p * t + (1.0 - p) * (1.0 - t)
    one_m = 1.0 - p_t
    loss = ce * (one_m * one_m)
    alpha_t = _ALPHA * t + (1.0 - _ALPHA) * (1.0 - t)
    loss = alpha_t * loss + 0.0
    key = lax.bitcast_convert_type(loss, jnp.int32)
    key_ref[0, 0] = jnp.where(mask_ref[...] == 0, key, -1)


def _loss_keys(predictions, targets, mask_plane, interpret=False):
    return pl.pallas_call(
        _loss_body,
        grid=(_B, _C),
        in_specs=[
            pl.BlockSpec((1, 1, _H, _W), lambda b, c: (b, c, 0, 0)),
            pl.BlockSpec((1, 1, _H, _W), lambda b, c: (b, c + 1, 0, 0)),
            pl.BlockSpec((_H, _W), lambda b, c: (0, 0)),
        ],
        out_specs=pl.BlockSpec((1, 1, _H, _W), lambda b, c: (b, c, 0, 0)),
        out_shape=jax.ShapeDtypeStruct((_B, _C, _H, _W), jnp.int32),
        interpret=interpret,
    )(predictions, targets, mask_plane)


def _make_hist_pass(stage, interpret=False):
    mesh = plsc.VectorSubcoreMesh(
        core_axis_name="c", subcore_axis_name="s",
        num_cores=2, num_subcores=16)

    @functools.partial(
        pl.kernel,
        out_type=(
            jax.ShapeDtypeStruct((_NW, _NB), jnp.int32),
            jax.ShapeDtypeStruct((_NW, _NB), jnp.float32),
        ),
        mesh=mesh,
        scratch_types=[
            pltpu.VMEM((_CHUNK,), jnp.int32),
            pltpu.VMEM((_CHUNK,), jnp.int32),
            pltpu.VMEM((_NB,), jnp.int32),
            pltpu.VMEM((_NB,), jnp.float32),
            pltpu.VMEM((16,), jnp.int32),
            pltpu.SemaphoreType.DMA,
            pltpu.SemaphoreType.DMA,
        ],
        compiler_params=pltpu.CompilerParams(needs_layout_passes=False),
        interpret=interpret,
    )
    def hist(keys_hbm, pfx_hbm, cnt_hbm, sum_hbm,
             buf0, buf1, cnt, sm, pfxv, sem0, sem1):
        wid = lax.axis_index("s") * 2 + lax.axis_index("c")
        base = wid * _PER_W

        zero16i = jnp.zeros((16,), jnp.int32)
        zero16f = jnp.zeros((16,), jnp.float32)

        def zbody(i, carry):
            cnt[pl.ds(i * 16, 16)] = zero16i
            sm[pl.ds(i * 16, 16)] = zero16f
            return carry

        lax.fori_loop(0, _NB // 16, zbody, 0)

        pltpu.sync_copy(pfx_hbm, pfxv)
        pfx = pfxv[...]
        ones = jnp.ones((16,), jnp.int32)

        bufs = (buf0, buf1)
        sems = (sem0, sem1)
        copies = [None, None]
        copies[0] = pltpu.async_copy(
            keys_hbm.at[pl.ds(base, _CHUNK)], buf0, sem0)
        for c in range(_NCHUNK):
            if c + 1 < _NCHUNK:
                nxt = (c + 1) % 2
                copies[nxt] = pltpu.async_copy(
                    keys_hbm.at[pl.ds(base + (c + 1) * _CHUNK, _CHUNK)],
                    bufs[nxt], sems[nxt])
            copies[c % 2].wait()
            buf = bufs[c % 2]

            def body(i, carry):
                key = buf[pl.ds(i * 16, 16)]
                if stage == 0:
                    valid = key >= 0
                    bucket = lax.shift_right_logical(key, 19)
                else:
                    valid = jnp.logical_and(
                        key >= 0, lax.shift_right_logical(key, 19) == pfx)
                    bucket = jnp.bitwise_and(
                        lax.shift_right_logical(key, 7), _NB - 1)
                bucket = jnp.where(valid, bucket, 0)
                plsc.addupdate_scatter(cnt, [bucket], ones, mask=valid)
                val = jnp.where(valid, plsc.bitcast(key, jnp.float32), 0.0)
                plsc.addupdate_scatter(sm, [bucket], val, mask=valid)
                return carry

            lax.fori_loop(0, _CHUNK // 16, body, 0)

        pltpu.sync_copy(cnt, cnt_hbm.at[wid])
        pltpu.sync_copy(sm, sum_hbm.at[wid])

    return hist


_make_hist_pass = functools.lru_cache(maxsize=None)(_make_hist_pass)


def _select(cnt, sm, need):
    rc = jnp.cumsum(cnt[::-1])[::-1]
    ca = rc - cnt
    cross = jnp.logical_and(ca < need, rc >= need)
    b = jnp.argmax(cross)
    rs = jnp.cumsum(sm[::-1])[::-1]
    sa = rs - sm
    return b, need - ca[b], sa[b]


def kernel(predictions, targets, batch_idx):
    mask_plane = lax.dynamic_index_in_dim(
        targets, batch_idx, axis=0, keepdims=False)[0]
    keys = _loss_keys(predictions, targets, mask_plane).reshape(_N)

    pfx0 = jnp.zeros((16,), jnp.int32)
    cnt_t, sum_t = _make_hist_pass(0)(keys, pfx0)
    cnt1 = cnt_t.sum(0)
    sum1 = sum_t.sum(0)
    b1, k1, s1 = _select(cnt1, sum1, _K)

    pfx1 = jnp.full((16,), b1, jnp.int32)
    cnt_t2, sum_t2 = _make_hist_pass(1)(keys, pfx1)
    cnt2 = cnt_t2.sum(0)
    sum2 = sum_t2.sum(0)
    b2, k2, s2 = _select(cnt2, sum2, k1)

    avg = sum2[b2] / cnt2[b2].astype(jnp.float32)
    res = (s1 + s2 + k2.astype(jnp.float32) * avg) / jnp.float32(_K)
    total = cnt1.sum()
    return jnp.where(total >= _K, res, -jnp.inf).astype(jnp.float32)
